# baseline (speedup 1.0000x reference)
import functools

import jax
import jax.numpy as jnp
from jax import lax
from jax.experimental import pallas as pl
from jax.experimental.pallas import tpu as pltpu
from jax.experimental.pallas import tpu_sc as plsc

_CB = 128
_KG = 5
_KS = 2


def _sc_gather(Np, Ep, D, NC, NS, epw):
    mesh = plsc.VectorSubcoreMesh(core_axis_name="c", subcore_axis_name="s")
    nch = epw // _CB
    nit = nch // _KG

    @functools.partial(
        pl.kernel,
        mesh=mesh,
        compiler_params=pltpu.CompilerParams(use_tc_tiling_on_sc=True),
        out_type=jax.ShapeDtypeStruct((Ep, D), jnp.float32),
        scratch_types=[
            pltpu.VMEM((nch, _CB), jnp.int32),
            pltpu.VMEM((_KG, _CB, D), jnp.float32),
            pltpu.SemaphoreType.DMA,
        ]
        + [pltpu.SemaphoreType.DMA] * _KG
        + [pltpu.SemaphoreType.DMA] * _KG,
    )
    def gk(A_hbm, row3_hbm, out_hbm, ridx, bufs, isem,
           g0, g1, g2, g3, g4, s0, s1, s2, s3, s4):
        gsems = (g0, g1, g2, g3, g4)
        ssems = (s0, s1, s2, s3, s4)
        wid = lax.axis_index("s") * NC + lax.axis_index("c")
        base = wid * epw
        pltpu.async_copy(row3_hbm.at[wid], ridx, isem).wait()

        def outer(it, carry):
            gbase = it * _KG
            gh = [
                pltpu.async_copy(A_hbm.at[ridx.at[gbase + k]], bufs.at[k], gsems[k])
                for k in range(_KG)
            ]
            sh = []
            for k in range(_KG):
                gh[k].wait()
                sh.append(
                    pltpu.async_copy(
                        bufs.at[k],
                        out_hbm.at[pl.ds(base + (gbase + k) * _CB, _CB)],
                        ssems[k],
                    )
                )
            for k in range(_KG):
                sh[k].wait()
            return carry

        lax.fori_loop(0, nit, outer, 0)

    return gk


def _sc_scatter(Np, Ep, D, NC, NS, epw):
    mesh = plsc.VectorSubcoreMesh(core_axis_name="c", subcore_axis_name="s")
    nch = epw // _CB
    nit = nch // _KS
    rpt = Np // NS

    @functools.partial(
        pl.kernel,
        mesh=mesh,
        compiler_params=pltpu.CompilerParams(use_tc_tiling_on_sc=True),
        out_type=(
            jax.ShapeDtypeStruct((NC * Np, D), jnp.float32),
            jax.ShapeDtypeStruct((NC * Np,), jnp.float32),
        ),
        scratch_types=[
            pltpu.VMEM((nch, _CB), jnp.int32),
            pltpu.VMEM((_KS, _CB, D), jnp.float32),
            pltpu.VMEM((_CB,), jnp.float32),
            pltpu.VMEM_SHARED((Np, D), jnp.float32),
            pltpu.VMEM_SHARED((Np,), jnp.float32),
            pltpu.SemaphoreType.DMA,
        ]
        + [pltpu.SemaphoreType.DMA] * _KS
        + [pltpu.SemaphoreType.DMA] * _KS
        + [pltpu.SemaphoreType.DMA] * _KS,
    )
    def sk(R_hbm, col3_hbm, z2_hbm, z1_hbm, S_hbm, C_hbm,
           cidx, rbufs, ones_v, acc, cacc, isem,
           r0s, r1s, a0, a1, c0, c1):
        rsems = (r0s, r1s)
        asems = (a0, a1)
        csems = (c0, c1)
        cid = lax.axis_index("c")
        sid = lax.axis_index("s")
        wid = sid * NC + cid
        base = wid * epw
        r0 = sid * rpt
        pltpu.async_copy(col3_hbm.at[wid], cidx, isem)
        pltpu.sync_copy(z2_hbm.at[pl.ds(r0, rpt)], acc.at[pl.ds(r0, rpt)])
        pltpu.sync_copy(z1_hbm.at[pl.ds(r0, rpt)], cacc.at[pl.ds(r0, rpt)])
        for k in range(_CB // 16):
            ones_v[pl.ds(k * 16, 16)] = jnp.ones((16,), jnp.float32)
        pltpu.make_async_copy(col3_hbm.at[wid], cidx, isem).wait()
        plsc.subcore_barrier()

        def outer(it, carry):
            gbase = it * _KS
            lh = [
                pltpu.async_copy(
                    R_hbm.at[pl.ds(base + (gbase + k) * _CB, _CB)],
                    rbufs.at[k],
                    rsems[k],
                )
                for k in range(_KS)
            ]
            ah = []
            ch = []
            for k in range(_KS):
                lh[k].wait()
                ah.append(
                    pltpu.async_copy(
                        rbufs.at[k], acc.at[cidx.at[gbase + k]], asems[k], add=True
                    )
                )
                ch.append(
                    pltpu.async_copy(
                        ones_v, cacc.at[cidx.at[gbase + k]], csems[k], add=True
                    )
                )
            for k in range(_KS):
                ah[k].wait()
                ch[k].wait()
            return carry

        lax.fori_loop(0, nit, outer, 0)
        plsc.subcore_barrier()
        pltpu.sync_copy(acc.at[pl.ds(r0, rpt)], S_hbm.at[pl.ds(cid * Np + r0, rpt)])
        pltpu.sync_copy(cacc.at[pl.ds(r0, rpt)], C_hbm.at[pl.ds(cid * Np + r0, rpt)])

    return sk


def kernel(x, edge_index, edge_attr, u, batch, W1, b1, W2, b2, W3, b3, W4, b4):
    N, D = x.shape
    E = edge_index.shape[1]
    G = u.shape[0]
    de = edge_attr.shape[1]

    try:
        info = plsc.get_sparse_core_info()
        NC, NS = info.num_cores, info.num_subcores
    except Exception:
        NC, NS = 2, 16
    NW = NC * NS

    BN = 1024
    Np = ((N + BN - 1) // BN) * BN
    unit = 2 * NW * _CB * _KG * _KS
    Ep = ((E + unit - 1) // unit) * unit
    BE = 4096

    row = jnp.asarray(edge_index[0], jnp.int32)
    col = jnp.asarray(edge_index[1], jnp.int32)
    pad_e = Ep - E
    pad_idx = N + (jnp.arange(pad_e, dtype=jnp.int32) % jnp.int32(max(Np - N, 1)))
    rowp = jnp.concatenate([row, pad_idx])
    colp = jnp.concatenate([col, pad_idx])
    eap = jnp.pad(edge_attr, ((0, pad_e), (0, 0)))
    xpad = jnp.pad(x, ((0, Np - N), (0, 0)))
    batchp = jnp.pad(batch.astype(jnp.int32), (0, Np - N)).reshape(Np, 1)

    b1r = b1.reshape(1, D)
    b2r = b2.reshape(1, D)
    b3r = b3.reshape(1, D)
    b4r = b4.reshape(1, D)

    def prep_fn(x_ref, W1_ref, b1_ref, W2_ref, W3_ref, b2_ref, u_ref, b3_ref,
                A_ref, W23_ref, bb_ref, uc_ref):
        W1a = W1_ref[:D, :]
        A_ref[...] = jnp.dot(x_ref[...], W1a, preferred_element_type=jnp.float32) + b1_ref[...]
        W3b = W3_ref[D:2 * D, :]
        W23_ref[...] = jnp.dot(W2_ref[...], W3b, preferred_element_type=jnp.float32)
        bb_ref[...] = jnp.dot(b2_ref[...], W3b, preferred_element_type=jnp.float32)
        uc_ref[...] = jnp.dot(u_ref[...], W3_ref[2 * D:, :], preferred_element_type=jnp.float32) + b3_ref[...]

    A, W23, bb, uc = pl.pallas_call(
        prep_fn,
        out_shape=(
            jax.ShapeDtypeStruct((Np, D), jnp.float32),
            jax.ShapeDtypeStruct((D, D), jnp.float32),
            jax.ShapeDtypeStruct((1, D), jnp.float32),
            jax.ShapeDtypeStruct((G, D), jnp.float32),
        ),
    )(xpad, W1, b1r, W2, W3, b2r, u, b3r)

    Eh = Ep // 2
    eph = Eh // NW
    nchh = eph // _CB
    gath = _sc_gather(Np, Eh, D, NC, NS, eph)
    scat = _sc_scatter(Np, Eh, D, NC, NS, eph)
    z2 = jnp.zeros((Np, D), jnp.float32)
    z1 = jnp.zeros((Np,), jnp.float32)

    def edge_fn(G_ref, ea_ref, W1_ref, R_ref):
        W1b = W1_ref[D:, :]
        R_ref[...] = jnp.maximum(
            G_ref[...] + jnp.dot(ea_ref[...], W1b, preferred_element_type=jnp.float32), 0.0)

    def edge_call(Gh, eahalf):
        return pl.pallas_call(
            edge_fn,
            grid=(Eh // BE,),
            in_specs=[
                pl.BlockSpec((BE, D), lambda i: (i, 0)),
                pl.BlockSpec((BE, de), lambda i: (i, 0)),
                pl.BlockSpec((D + de, D), lambda i: (0, 0)),
            ],
            out_specs=pl.BlockSpec((BE, D), lambda i: (i, 0)),
            out_shape=jax.ShapeDtypeStruct((Eh, D), jnp.float32),
        )(Gh, eahalf, W1)

    rowh = rowp.reshape(2, NW, nchh, _CB)
    colh = colp.reshape(2, NW, nchh, _CB)
    eah = eap.reshape(2, Eh, de)
    G0 = gath(A, rowh[0])
    G1 = gath(A, rowh[1])
    R0 = edge_call(G0, eah[0])
    R1 = edge_call(G1, eah[1])
    Sa2, Ca2 = scat(R0, colh[0], z2, z1)
    Sb2, Cb2 = scat(R1, colh[1], z2, z1)
    S = Sa2.reshape(NC, Np, D)
    C = Ca2.reshape(NC, Np, 1)
    Sb = Sb2.reshape(NC, Np, D)
    Cb = Cb2.reshape(NC, Np, 1)

    def node_fn(x_ref, S_ref, Sb_ref, C_ref, Cb_ref, b_ref, W3_ref, W23_ref,
                bb_ref, uc_ref, W4_ref, b4_ref, o_ref):
        S01 = S_ref[0] + S_ref[1] + Sb_ref[0] + Sb_ref[1]
        cnt = C_ref[0] + C_ref[1] + Cb_ref[0] + Cb_ref[1]
        inv = 1.0 / jnp.maximum(cnt, 1.0)
        m0 = jnp.minimum(cnt, 1.0)
        oneh = (b_ref[...] == lax.broadcasted_iota(jnp.int32, (BN, G), 1)).astype(jnp.float32)
        h = (jnp.dot(x_ref[...], W3_ref[:D, :], preferred_element_type=jnp.float32)
             + jnp.dot(S01 * inv, W23_ref[...], preferred_element_type=jnp.float32)
             + m0 * bb_ref[...]
             + jnp.dot(oneh, uc_ref[...], preferred_element_type=jnp.float32))
        o_ref[...] = jnp.dot(jnp.maximum(h, 0.0), W4_ref[...], preferred_element_type=jnp.float32) + b4_ref[...]

    out = pl.pallas_call(
        node_fn,
        grid=(Np // BN,),
        in_specs=[
            pl.BlockSpec((BN, D), lambda i: (i, 0)),
            pl.BlockSpec((NC, BN, D), lambda i: (0, i, 0)),
            pl.BlockSpec((NC, BN, D), lambda i: (0, i, 0)),
            pl.BlockSpec((NC, BN, 1), lambda i: (0, i, 0)),
            pl.BlockSpec((NC, BN, 1), lambda i: (0, i, 0)),
            pl.BlockSpec((BN, 1), lambda i: (i, 0)),
            pl.BlockSpec((2 * D + u.shape[1], D), lambda i: (0, 0)),
            pl.BlockSpec((D, D), lambda i: (0, 0)),
            pl.BlockSpec((1, D), lambda i: (0, 0)),
            pl.BlockSpec((G, D), lambda i: (0, 0)),
            pl.BlockSpec((D, D), lambda i: (0, 0)),
            pl.BlockSpec((1, D), lambda i: (0, 0)),
        ],
        out_specs=pl.BlockSpec((BN, D), lambda i: (i, 0)),
        out_shape=jax.ShapeDtypeStruct((Np, D), jnp.float32),
    )(xpad, S, Sb, C, Cb, batchp, W3, W23, bb, uc, W4, b4r)

    return out[:N]


# speedup vs baseline: 1.1030x; 1.1030x over previous
import functools

import jax
import jax.numpy as jnp
from jax import lax
from jax.experimental import pallas as pl
from jax.experimental.pallas import tpu as pltpu
from jax.experimental.pallas import tpu_sc as plsc

_CB = 128
_KG = 5
_KS = 2


def _sc_gather(Np, Ep, D, NC, NS, epw):
    mesh = plsc.VectorSubcoreMesh(core_axis_name="c", subcore_axis_name="s")
    nch = epw // _CB
    nit = nch // _KG

    @functools.partial(
        pl.kernel,
        mesh=mesh,
        compiler_params=pltpu.CompilerParams(use_tc_tiling_on_sc=True),
        out_type=jax.ShapeDtypeStruct((Ep, D), jnp.float32),
        scratch_types=[
            pltpu.VMEM((nch, _CB), jnp.int32),
            pltpu.VMEM((_KG, _CB, D), jnp.float32),
            pltpu.SemaphoreType.DMA,
        ]
        + [pltpu.SemaphoreType.DMA] * _KG
        + [pltpu.SemaphoreType.DMA] * _KG,
    )
    def gk(A_hbm, row3_hbm, out_hbm, ridx, bufs, isem,
           g0, g1, g2, g3, g4, s0, s1, s2, s3, s4):
        gsems = (g0, g1, g2, g3, g4)
        ssems = (s0, s1, s2, s3, s4)
        wid = lax.axis_index("s") * NC + lax.axis_index("c")
        base = wid * epw
        pltpu.async_copy(row3_hbm.at[wid], ridx, isem).wait()

        def outer(it, carry):
            gbase = it * _KG
            gh = [
                pltpu.async_copy(A_hbm.at[ridx.at[gbase + k]], bufs.at[k], gsems[k])
                for k in range(_KG)
            ]
            sh = []
            for k in range(_KG):
                gh[k].wait()
                sh.append(
                    pltpu.async_copy(
                        bufs.at[k],
                        out_hbm.at[pl.ds(base + (gbase + k) * _CB, _CB)],
                        ssems[k],
                    )
                )
            for k in range(_KG):
                sh[k].wait()
            return carry

        lax.fori_loop(0, nit, outer, 0)

    return gk


def _sc_scatter(Np, Ep, D, NC, NS, epw):
    mesh = plsc.VectorSubcoreMesh(core_axis_name="c", subcore_axis_name="s")
    nch = epw // _CB
    nit = nch // _KS
    rpt = Np // NS

    @functools.partial(
        pl.kernel,
        mesh=mesh,
        compiler_params=pltpu.CompilerParams(use_tc_tiling_on_sc=True),
        out_type=(
            jax.ShapeDtypeStruct((NC * Np, D), jnp.float32),
            jax.ShapeDtypeStruct((NC * Np,), jnp.float32),
        ),
        scratch_types=[
            pltpu.VMEM((nch, _CB), jnp.int32),
            pltpu.VMEM((_KS, _CB, D), jnp.float32),
            pltpu.VMEM((_CB,), jnp.float32),
            pltpu.VMEM_SHARED((Np, D), jnp.float32),
            pltpu.VMEM_SHARED((Np,), jnp.float32),
            pltpu.SemaphoreType.DMA,
        ]
        + [pltpu.SemaphoreType.DMA] * _KS
        + [pltpu.SemaphoreType.DMA] * _KS
        + [pltpu.SemaphoreType.DMA] * _KS,
    )
    def sk(R_hbm, col3_hbm, z2_hbm, z1_hbm, S_hbm, C_hbm,
           cidx, rbufs, ones_v, acc, cacc, isem,
           r0s, r1s, a0, a1, c0, c1):
        rsems = (r0s, r1s)
        asems = (a0, a1)
        csems = (c0, c1)
        cid = lax.axis_index("c")
        sid = lax.axis_index("s")
        wid = sid * NC + cid
        base = wid * epw
        r0 = sid * rpt
        pltpu.async_copy(col3_hbm.at[wid], cidx, isem)
        pltpu.sync_copy(z2_hbm.at[pl.ds(r0, rpt)], acc.at[pl.ds(r0, rpt)])
        pltpu.sync_copy(z1_hbm.at[pl.ds(r0, rpt)], cacc.at[pl.ds(r0, rpt)])
        for k in range(_CB // 16):
            ones_v[pl.ds(k * 16, 16)] = jnp.ones((16,), jnp.float32)
        pltpu.make_async_copy(col3_hbm.at[wid], cidx, isem).wait()
        plsc.subcore_barrier()

        def outer(it, carry):
            gbase = it * _KS
            lh = [
                pltpu.async_copy(
                    R_hbm.at[pl.ds(base + (gbase + k) * _CB, _CB)],
                    rbufs.at[k],
                    rsems[k],
                )
                for k in range(_KS)
            ]
            ah = []
            ch = []
            for k in range(_KS):
                lh[k].wait()
                ah.append(
                    pltpu.async_copy(
                        rbufs.at[k], acc.at[cidx.at[gbase + k]], asems[k], add=True
                    )
                )
                ch.append(
                    pltpu.async_copy(
                        ones_v, cacc.at[cidx.at[gbase + k]], csems[k], add=True
                    )
                )
            for k in range(_KS):
                ah[k].wait()
                ch[k].wait()
            return carry

        lax.fori_loop(0, nit, outer, 0)
        plsc.subcore_barrier()
        pltpu.sync_copy(acc.at[pl.ds(r0, rpt)], S_hbm.at[pl.ds(cid * Np + r0, rpt)])
        pltpu.sync_copy(cacc.at[pl.ds(r0, rpt)], C_hbm.at[pl.ds(cid * Np + r0, rpt)])

    return sk


def kernel(x, edge_index, edge_attr, u, batch, W1, b1, W2, b2, W3, b3, W4, b4):
    N, D = x.shape
    E = edge_index.shape[1]
    G = u.shape[0]
    de = edge_attr.shape[1]

    try:
        info = plsc.get_sparse_core_info()
        NC, NS = info.num_cores, info.num_subcores
    except Exception:
        NC, NS = 2, 16
    NW = NC * NS

    BN = 1024
    Np = ((N + BN - 1) // BN) * BN
    unit = NW * _CB * _KG * _KS
    Ep = ((E + unit - 1) // unit) * unit
    BE = 4096

    row = jnp.asarray(edge_index[0], jnp.int32)
    col = jnp.asarray(edge_index[1], jnp.int32)
    pad_e = Ep - E
    pad_idx = N + (jnp.arange(pad_e, dtype=jnp.int32) % jnp.int32(max(Np - N, 1)))
    rowp = jnp.concatenate([row, pad_idx])
    colp = jnp.concatenate([col, pad_idx])
    eap = jnp.pad(edge_attr, ((0, pad_e), (0, 0)))
    xpad = jnp.pad(x, ((0, Np - N), (0, 0)))
    batchp = jnp.pad(batch.astype(jnp.int32), (0, Np - N)).reshape(Np, 1)

    b1r = b1.reshape(1, D)
    b2r = b2.reshape(1, D)
    b3r = b3.reshape(1, D)
    b4r = b4.reshape(1, D)

    def prep_fn(x_ref, W1_ref, b1_ref, W2_ref, W3_ref, b2_ref, u_ref, b3_ref,
                A_ref, W23_ref, bb_ref, uc_ref):
        W1a = W1_ref[:D, :]
        A_ref[...] = jnp.dot(x_ref[...], W1a, preferred_element_type=jnp.float32) + b1_ref[...]
        W3b = W3_ref[D:2 * D, :]
        W23_ref[...] = jnp.dot(W2_ref[...], W3b, preferred_element_type=jnp.float32)
        bb_ref[...] = jnp.dot(b2_ref[...], W3b, preferred_element_type=jnp.float32)
        uc_ref[...] = jnp.dot(u_ref[...], W3_ref[2 * D:, :], preferred_element_type=jnp.float32) + b3_ref[...]

    A, W23, bb, uc = pl.pallas_call(
        prep_fn,
        out_shape=(
            jax.ShapeDtypeStruct((Np, D), jnp.float32),
            jax.ShapeDtypeStruct((D, D), jnp.float32),
            jax.ShapeDtypeStruct((1, D), jnp.float32),
            jax.ShapeDtypeStruct((G, D), jnp.float32),
        ),
    )(xpad, W1, b1r, W2, W3, b2r, u, b3r)

    epw = Ep // NW
    nch = epw // _CB
    row3 = rowp.reshape(NW, nch, _CB)
    Gm = _sc_gather(Np, Ep, D, NC, NS, epw)(A, row3)

    def edge_fn(G_ref, ea_ref, W1_ref, R_ref):
        W1b = W1_ref[D:, :]
        R_ref[...] = jnp.maximum(
            G_ref[...] + jnp.dot(ea_ref[...], W1b, preferred_element_type=jnp.float32), 0.0)

    R = pl.pallas_call(
        edge_fn,
        grid=(Ep // BE,),
        in_specs=[
            pl.BlockSpec((BE, D), lambda i: (i, 0)),
            pl.BlockSpec((BE, de), lambda i: (i, 0)),
            pl.BlockSpec((D + de, D), lambda i: (0, 0)),
        ],
        out_specs=pl.BlockSpec((BE, D), lambda i: (i, 0)),
        out_shape=jax.ShapeDtypeStruct((Ep, D), jnp.float32),
    )(Gm, eap, W1)

    z2 = jnp.zeros((Np, D), jnp.float32)
    z1 = jnp.zeros((Np,), jnp.float32)
    col3 = colp.reshape(NW, nch, _CB)
    S2, C2 = _sc_scatter(Np, Ep, D, NC, NS, epw)(R, col3, z2, z1)
    S = S2.reshape(NC, Np, D)
    C = C2.reshape(NC, Np, 1)

    def node_fn(x_ref, S_ref, C_ref, b_ref, W3_ref, W23_ref,
                bb_ref, uc_ref, W4_ref, b4_ref, o_ref):
        S01 = S_ref[0] + S_ref[1]
        cnt = C_ref[0] + C_ref[1]
        inv = 1.0 / jnp.maximum(cnt, 1.0)
        m0 = jnp.minimum(cnt, 1.0)
        oneh = (b_ref[...] == lax.broadcasted_iota(jnp.int32, (BN, G), 1)).astype(jnp.float32)
        h = (jnp.dot(x_ref[...], W3_ref[:D, :], preferred_element_type=jnp.float32)
             + jnp.dot(S01 * inv, W23_ref[...], preferred_element_type=jnp.float32)
             + m0 * bb_ref[...]
             + jnp.dot(oneh, uc_ref[...], preferred_element_type=jnp.float32))
        o_ref[...] = jnp.dot(jnp.maximum(h, 0.0), W4_ref[...], preferred_element_type=jnp.float32) + b4_ref[...]

    out = pl.pallas_call(
        node_fn,
        grid=(Np // BN,),
        in_specs=[
            pl.BlockSpec((BN, D), lambda i: (i, 0)),
            pl.BlockSpec((NC, BN, D), lambda i: (0, i, 0)),
            pl.BlockSpec((NC, BN, 1), lambda i: (0, i, 0)),
            pl.BlockSpec((BN, 1), lambda i: (i, 0)),
            pl.BlockSpec((2 * D + u.shape[1], D), lambda i: (0, 0)),
            pl.BlockSpec((D, D), lambda i: (0, 0)),
            pl.BlockSpec((1, D), lambda i: (0, 0)),
            pl.BlockSpec((G, D), lambda i: (0, 0)),
            pl.BlockSpec((D, D), lambda i: (0, 0)),
            pl.BlockSpec((1, D), lambda i: (0, 0)),
        ],
        out_specs=pl.BlockSpec((BN, D), lambda i: (i, 0)),
        out_shape=jax.ShapeDtypeStruct((Np, D), jnp.float32),
    )(xpad, S, C, batchp, W3, W23, bb, uc, W4, b4r)

    return out[:N]


# speedup vs baseline: 1.1035x; 1.0004x over previous
import functools

import jax
import jax.numpy as jnp
from jax import lax
from jax.experimental import pallas as pl
from jax.experimental.pallas import tpu as pltpu
from jax.experimental.pallas import tpu_sc as plsc

_CB = 128
_KG = 5
_KS = 2


def _sc_gather(Np, Ep, D, NC, NS, epw):
    mesh = plsc.VectorSubcoreMesh(core_axis_name="c", subcore_axis_name="s")
    nch = epw // _CB
    nit = nch // _KG

    @functools.partial(
        pl.kernel,
        mesh=mesh,
        compiler_params=pltpu.CompilerParams(use_tc_tiling_on_sc=True),
        out_type=jax.ShapeDtypeStruct((Ep, D), jnp.float32),
        scratch_types=[
            pltpu.VMEM((nch, _CB), jnp.int32),
            pltpu.VMEM((_KG, _CB, D), jnp.float32),
            pltpu.SemaphoreType.DMA,
        ]
        + [pltpu.SemaphoreType.DMA] * _KG
        + [pltpu.SemaphoreType.DMA] * _KG,
    )
    def gk(A_hbm, row3_hbm, out_hbm, ridx, bufs, isem,
           g0, g1, g2, g3, g4, s0, s1, s2, s3, s4):
        gsems = (g0, g1, g2, g3, g4)
        ssems = (s0, s1, s2, s3, s4)
        wid = lax.axis_index("s") * NC + lax.axis_index("c")
        base = wid * epw
        pltpu.async_copy(row3_hbm.at[wid], ridx, isem).wait()

        def outer(it, carry):
            gbase = it * _KG
            gh = [
                pltpu.async_copy(A_hbm.at[ridx.at[gbase + k]], bufs.at[k], gsems[k])
                for k in range(_KG)
            ]
            sh = []
            for k in range(_KG):
                gh[k].wait()
                sh.append(
                    pltpu.async_copy(
                        bufs.at[k],
                        out_hbm.at[pl.ds(base + (gbase + k) * _CB, _CB)],
                        ssems[k],
                    )
                )
            for k in range(_KG):
                sh[k].wait()
            return carry

        lax.fori_loop(0, nit, outer, 0)

    return gk


def _sc_scatter(Np, Ep, D, NC, NS, epw):
    mesh = plsc.VectorSubcoreMesh(core_axis_name="c", subcore_axis_name="s")
    nch = epw // _CB
    nit = nch // _KS
    rpt = Np // NS

    @functools.partial(
        pl.kernel,
        mesh=mesh,
        compiler_params=pltpu.CompilerParams(use_tc_tiling_on_sc=True),
        out_type=(
            jax.ShapeDtypeStruct((NC * Np, D), jnp.float32),
            jax.ShapeDtypeStruct((NC * Np,), jnp.float32),
        ),
        scratch_types=[
            pltpu.VMEM((nch, _CB), jnp.int32),
            pltpu.VMEM((_KS, _CB, D), jnp.float32),
            pltpu.VMEM((_CB,), jnp.float32),
            pltpu.VMEM_SHARED((Np, D), jnp.float32),
            pltpu.VMEM_SHARED((Np,), jnp.float32),
            pltpu.SemaphoreType.DMA,
        ]
        + [pltpu.SemaphoreType.DMA] * _KS
        + [pltpu.SemaphoreType.DMA] * _KS
        + [pltpu.SemaphoreType.DMA] * _KS,
    )
    def sk(R_hbm, col3_hbm, z2_hbm, z1_hbm, S_hbm, C_hbm,
           cidx, rbufs, ones_v, acc, cacc, isem,
           r0s, r1s, a0, a1, c0, c1):
        rsems = (r0s, r1s)
        asems = (a0, a1)
        csems = (c0, c1)
        cid = lax.axis_index("c")
        sid = lax.axis_index("s")
        wid = sid * NC + cid
        base = wid * epw
        r0 = sid * rpt
        pltpu.async_copy(col3_hbm.at[wid], cidx, isem)
        pltpu.sync_copy(z2_hbm.at[pl.ds(r0, rpt)], acc.at[pl.ds(r0, rpt)])
        pltpu.sync_copy(z1_hbm.at[pl.ds(r0, rpt)], cacc.at[pl.ds(r0, rpt)])
        for k in range(_CB // 16):
            ones_v[pl.ds(k * 16, 16)] = jnp.ones((16,), jnp.float32)
        pltpu.make_async_copy(col3_hbm.at[wid], cidx, isem).wait()
        plsc.subcore_barrier()

        def outer(it, carry):
            gbase = it * _KS
            lh = [
                pltpu.async_copy(
                    R_hbm.at[pl.ds(base + (gbase + k) * _CB, _CB)],
                    rbufs.at[k],
                    rsems[k],
                )
                for k in range(_KS)
            ]
            ah = []
            ch = []
            for k in range(_KS):
                lh[k].wait()
                ah.append(
                    pltpu.async_copy(
                        rbufs.at[k], acc.at[cidx.at[gbase + k]], asems[k], add=True
                    )
                )
                ch.append(
                    pltpu.async_copy(
                        ones_v, cacc.at[cidx.at[gbase + k]], csems[k], add=True
                    )
                )
            for k in range(_KS):
                ah[k].wait()
                ch[k].wait()
            return carry

        lax.fori_loop(0, nit, outer, 0)
        plsc.subcore_barrier()
        pltpu.sync_copy(acc.at[pl.ds(r0, rpt)], S_hbm.at[pl.ds(cid * Np + r0, rpt)])
        pltpu.sync_copy(cacc.at[pl.ds(r0, rpt)], C_hbm.at[pl.ds(cid * Np + r0, rpt)])

    return sk


def kernel(x, edge_index, edge_attr, u, batch, W1, b1, W2, b2, W3, b3, W4, b4):
    N, D = x.shape
    E = edge_index.shape[1]
    G = u.shape[0]
    de = edge_attr.shape[1]

    try:
        info = plsc.get_sparse_core_info()
        NC, NS = info.num_cores, info.num_subcores
    except Exception:
        NC, NS = 2, 16
    NW = NC * NS

    BN = 1024
    Np = ((N + BN - 1) // BN) * BN
    unit = NW * _CB * _KG * _KS
    Ep = ((E + unit - 1) // unit) * unit
    BE = 8192

    row = jnp.asarray(edge_index[0], jnp.int32)
    col = jnp.asarray(edge_index[1], jnp.int32)
    pad_e = Ep - E
    pad_idx = N + (jnp.arange(pad_e, dtype=jnp.int32) % jnp.int32(max(Np - N, 1)))
    rowp = jnp.concatenate([row, pad_idx])
    colp = jnp.concatenate([col, pad_idx])
    eap = jnp.pad(edge_attr, ((0, pad_e), (0, 0)))
    xpad = jnp.pad(x, ((0, Np - N), (0, 0)))
    batchp = jnp.pad(batch.astype(jnp.int32), (0, Np - N)).reshape(Np, 1)

    b1r = b1.reshape(1, D)
    b2r = b2.reshape(1, D)
    b3r = b3.reshape(1, D)
    b4r = b4.reshape(1, D)

    def prep_fn(x_ref, W1_ref, b1_ref, W2_ref, W3_ref, b2_ref, u_ref, b3_ref,
                A_ref, W23_ref, bb_ref, uc_ref):
        W1a = W1_ref[:D, :]
        A_ref[...] = jnp.dot(x_ref[...], W1a, preferred_element_type=jnp.float32) + b1_ref[...]
        W3b = W3_ref[D:2 * D, :]
        W23_ref[...] = jnp.dot(W2_ref[...], W3b, preferred_element_type=jnp.float32)
        bb_ref[...] = jnp.dot(b2_ref[...], W3b, preferred_element_type=jnp.float32)
        uc_ref[...] = jnp.dot(u_ref[...], W3_ref[2 * D:, :], preferred_element_type=jnp.float32) + b3_ref[...]

    BP = 2048
    A, W23, bb, uc = pl.pallas_call(
        prep_fn,
        grid=(Np // BP,),
        in_specs=[
            pl.BlockSpec((BP, D), lambda i: (i, 0)),
            pl.BlockSpec((D + de, D), lambda i: (0, 0)),
            pl.BlockSpec((1, D), lambda i: (0, 0)),
            pl.BlockSpec((D, D), lambda i: (0, 0)),
            pl.BlockSpec((2 * D + u.shape[1], D), lambda i: (0, 0)),
            pl.BlockSpec((1, D), lambda i: (0, 0)),
            pl.BlockSpec((G, u.shape[1]), lambda i: (0, 0)),
            pl.BlockSpec((1, D), lambda i: (0, 0)),
        ],
        out_specs=(
            pl.BlockSpec((BP, D), lambda i: (i, 0)),
            pl.BlockSpec((D, D), lambda i: (0, 0)),
            pl.BlockSpec((1, D), lambda i: (0, 0)),
            pl.BlockSpec((G, D), lambda i: (0, 0)),
        ),
        out_shape=(
            jax.ShapeDtypeStruct((Np, D), jnp.float32),
            jax.ShapeDtypeStruct((D, D), jnp.float32),
            jax.ShapeDtypeStruct((1, D), jnp.float32),
            jax.ShapeDtypeStruct((G, D), jnp.float32),
        ),
    )(xpad, W1, b1r, W2, W3, b2r, u, b3r)

    epw = Ep // NW
    nch = epw // _CB
    row3 = rowp.reshape(NW, nch, _CB)
    Gm = _sc_gather(Np, Ep, D, NC, NS, epw)(A, row3)

    def edge_fn(G_ref, ea_ref, W1_ref, R_ref):
        W1b = W1_ref[D:, :]
        R_ref[...] = jnp.maximum(
            G_ref[...] + jnp.dot(ea_ref[...], W1b, preferred_element_type=jnp.float32), 0.0)

    R = pl.pallas_call(
        edge_fn,
        grid=(Ep // BE,),
        in_specs=[
            pl.BlockSpec((BE, D), lambda i: (i, 0)),
            pl.BlockSpec((BE, de), lambda i: (i, 0)),
            pl.BlockSpec((D + de, D), lambda i: (0, 0)),
        ],
        out_specs=pl.BlockSpec((BE, D), lambda i: (i, 0)),
        out_shape=jax.ShapeDtypeStruct((Ep, D), jnp.float32),
    )(Gm, eap, W1)

    z2 = jnp.zeros((Np, D), jnp.float32)
    z1 = jnp.zeros((Np,), jnp.float32)
    col3 = colp.reshape(NW, nch, _CB)
    S2, C2 = _sc_scatter(Np, Ep, D, NC, NS, epw)(R, col3, z2, z1)
    S = S2.reshape(NC, Np, D)
    C = C2.reshape(NC, Np, 1)

    def node_fn(x_ref, S_ref, C_ref, b_ref, W3_ref, W23_ref,
                bb_ref, uc_ref, W4_ref, b4_ref, o_ref):
        S01 = S_ref[0] + S_ref[1]
        cnt = C_ref[0] + C_ref[1]
        inv = 1.0 / jnp.maximum(cnt, 1.0)
        m0 = jnp.minimum(cnt, 1.0)
        oneh = (b_ref[...] == lax.broadcasted_iota(jnp.int32, (BN, G), 1)).astype(jnp.float32)
        h = (jnp.dot(x_ref[...], W3_ref[:D, :], preferred_element_type=jnp.float32)
             + jnp.dot(S01 * inv, W23_ref[...], preferred_element_type=jnp.float32)
             + m0 * bb_ref[...]
             + jnp.dot(oneh, uc_ref[...], preferred_element_type=jnp.float32))
        o_ref[...] = jnp.dot(jnp.maximum(h, 0.0), W4_ref[...], preferred_element_type=jnp.float32) + b4_ref[...]

    out = pl.pallas_call(
        node_fn,
        grid=(Np // BN,),
        in_specs=[
            pl.BlockSpec((BN, D), lambda i: (i, 0)),
            pl.BlockSpec((NC, BN, D), lambda i: (0, i, 0)),
            pl.BlockSpec((NC, BN, 1), lambda i: (0, i, 0)),
            pl.BlockSpec((BN, 1), lambda i: (i, 0)),
            pl.BlockSpec((2 * D + u.shape[1], D), lambda i: (0, 0)),
            pl.BlockSpec((D, D), lambda i: (0, 0)),
            pl.BlockSpec((1, D), lambda i: (0, 0)),
            pl.BlockSpec((G, D), lambda i: (0, 0)),
            pl.BlockSpec((D, D), lambda i: (0, 0)),
            pl.BlockSpec((1, D), lambda i: (0, 0)),
        ],
        out_specs=pl.BlockSpec((BN, D), lambda i: (i, 0)),
        out_shape=jax.ShapeDtypeStruct((Np, D), jnp.float32),
    )(xpad, S, C, batchp, W3, W23, bb, uc, W4, b4r)

    return out[:N]
